# Initial kernel scaffold; baseline (speedup 1.0000x reference)
import jax
import jax.numpy as jnp
from jax.experimental import pallas as pl


def kernel(t_local, y, W_theta, W1, W2, b16, b64, sup1_idx, sup1_val, sup2_idx, sup2_val):
    raise NotImplementedError("write your pallas kernel here")


# speedup vs baseline: 1.0029x; 1.0029x over previous
import jax
import jax.numpy as jnp
from jax.experimental import pallas as pl

N = 10000
DEG = 16
LAT = 16
UNITS = 64
K = 2
B = 16
NUM_MAT = 2 * K + 1


def _spmm(idx, val, x):
    return jnp.zeros((N, x.shape[1]), x.dtype).at[idx[0]].add(val[:, None] * x[idx[1]])


def _gconv(yflat, sups, W, b, out_size):
    bsz = yflat.shape[0]
    x = yflat.reshape(bsz, N, -1)
    insz = x.shape[2]
    x0 = jnp.transpose(x, (1, 2, 0)).reshape(N, insz * bsz)
    xs = [x0]
    for idx, val in sups:
        x1 = _spmm(idx, val, x0)
        xs.append(x1)
        xkm1, xkm2 = x1, x0
        for _ in range(2, K + 1):
            x2 = 2.0 * _spmm(idx, val, xkm1) - xkm2
            xs.append(x2)
            xkm1, xkm2 = x2, xkm1
    x = jnp.stack(xs).reshape(NUM_MAT, N, insz, bsz)
    x = jnp.transpose(x, (3, 1, 2, 0)).reshape(bsz * N, insz * NUM_MAT)
    x = x @ W + b
    return x.reshape(bsz, N * out_size)


def _finish(theta_pre_ref, c_pre_ref, o_ref):
    o_ref[...] = -jax.nn.sigmoid(theta_pre_ref[...]) * jnp.tanh(c_pre_ref[...])


def kernel(t_local, y, W_theta, W1, W2, b16, b64, sup1_idx, sup1_val, sup2_idx, sup2_val):
    sups = [(sup1_idx, sup1_val), (sup2_idx, sup2_val)]
    theta_pre = _gconv(y, sups, W_theta, b16, LAT)
    c = jnp.tanh(_gconv(y, sups, W1, b64, UNITS))
    c_pre = _gconv(c, sups, W2, b16, LAT)
    grid = 25
    blk = (B, N * LAT // grid)
    return pl.pallas_call(
        _finish,
        grid=(grid,),
        in_specs=[pl.BlockSpec(blk, lambda i: (0, i))] * 2,
        out_specs=pl.BlockSpec(blk, lambda i: (0, i)),
        out_shape=jax.ShapeDtypeStruct((B, N * LAT), jnp.float32),
    )(theta_pre, c_pre)


# speedup vs baseline: 3.3422x; 3.3325x over previous
import functools

import jax
import jax.numpy as jnp
from jax import lax
from jax.experimental import pallas as pl
from jax.experimental.pallas import tpu as pltpu
from jax.experimental.pallas import tpu_sc as plsc

N = 10000
DEG = 16
LAT = 16
UNITS = 64
B = 16
F = B * LAT
NC, NS, LN = 2, 16, 16
NT = NC * NS
NPAD = 10240
EPAD = NPAD * DEG
NPT = NPAD // NT
CN = 8
CE = CN * DEG
NCH = NPT // CN
FH = 128
NRT = NPAD // NS
M = NPAD * B
R = 2048
GRID = M // R


def _mesh():
    return plsc.VectorSubcoreMesh(
        core_axis_name="c", subcore_axis_name="s", num_cores=NC, num_subcores=NS
    )


def _t2_gather_body(cols_hbm, vals_hbm, x_hbm, out_hbm, idx_v, vals_v, rows_v,
                    out_v, sem):
    wid = lax.axis_index("s") * NC + lax.axis_index("c")
    nbase = wid * NPT

    def chunk(ch, carry):
        n0 = nbase + ch * CN
        e0 = n0 * DEG
        pltpu.sync_copy(cols_hbm.at[pl.ds(e0, CE)], idx_v)
        pltpu.sync_copy(vals_hbm.at[pl.ds(e0, CE)], vals_v)
        pltpu.async_copy(x_hbm.at[idx_v], rows_v, sem).wait()

        def node(n, c2):
            e = n * DEG
            accs = [jnp.zeros((LN,), jnp.float32) for _ in range(F // LN)]
            for j in range(DEG):
                vb = plsc.load_gather(
                    vals_v, [jnp.full((LN,), e + j, jnp.int32)])
                for c in range(F // LN):
                    accs[c] = accs[c] + vb * rows_v[e + j, pl.ds(c * LN, LN)]
            for c in range(F // LN):
                out_v[n, pl.ds(c * LN, LN)] = accs[c]
            return c2

        lax.fori_loop(0, CN, node, 0)
        pltpu.sync_copy(out_v, out_hbm.at[pl.ds(n0, CN)])
        return carry

    lax.fori_loop(0, NCH, chunk, 0)


def _t2_spmm(cols, vals, x):
    k = pl.kernel(
        _t2_gather_body,
        out_type=jax.ShapeDtypeStruct((NPAD, F), jnp.float32),
        mesh=_mesh(),
        compiler_params=pltpu.CompilerParams(needs_layout_passes=False),
        scratch_types=[
            pltpu.VMEM((CE,), jnp.int32),
            pltpu.VMEM((CE,), jnp.float32),
            pltpu.VMEM((CE, F), jnp.float32),
            pltpu.VMEM((CN, F), jnp.float32),
            pltpu.SemaphoreType.DMA,
        ],
    )
    return k(cols, vals, x)


def _t1_scatter_body(two_parts, dst_hbm, vals_hbm, x_hbm, out_hbm, *rest):
    if two_parts:
        idx_v, vals_v, xrow_a, xrow_b, block_v, zbuf, spmem = rest
    else:
        idx_v, vals_v, xrow_a, block_v, zbuf, spmem = rest
        xrow_b = None
    cid = lax.axis_index("c")
    sid = lax.axis_index("s")
    wid = sid * NC + cid
    nbase = wid * NPT
    rows0 = sid * NRT

    def zrow(i, c2):
        for c in range(FH // LN):
            zbuf[i, pl.ds(c * LN, LN)] = jnp.zeros((LN,), jnp.float32)
        return c2

    lax.fori_loop(0, FH, zrow, 0)

    for p in range(2):
        def zcp(kk, c2):
            pltpu.sync_copy(zbuf, spmem.at[pl.ds(rows0 + kk * FH, FH)])
            return c2

        lax.fori_loop(0, NRT // FH, zcp, 0)
        plsc.subcore_barrier()

        def chunk(ch, c2):
            n0 = nbase + ch * CN
            e0 = n0 * DEG
            pltpu.sync_copy(dst_hbm.at[pl.ds(e0, CE)], idx_v)
            pltpu.sync_copy(vals_hbm.at[pl.ds(e0, CE)], vals_v)
            if two_parts:
                pltpu.sync_copy(
                    x_hbm.at[0, pl.ds(n0, CN), pl.ds(p * FH, FH)], xrow_a)
                pltpu.sync_copy(
                    x_hbm.at[1, pl.ds(n0, CN), pl.ds(p * FH, FH)], xrow_b)
            else:
                pltpu.sync_copy(
                    x_hbm.at[pl.ds(n0, CN), pl.ds(p * FH, FH)], xrow_a)

            def node(n, c3):
                e = n * DEG
                if two_parts:
                    rs = [xrow_a[n, pl.ds(c * LN, LN)]
                          + xrow_b[n, pl.ds(c * LN, LN)]
                          for c in range(FH // LN)]
                else:
                    rs = [xrow_a[n, pl.ds(c * LN, LN)]
                          for c in range(FH // LN)]
                for j in range(DEG):
                    vb = plsc.load_gather(
                        vals_v, [jnp.full((LN,), e + j, jnp.int32)])
                    for c in range(FH // LN):
                        block_v[e + j, pl.ds(c * LN, LN)] = vb * rs[c]
                return c3

            lax.fori_loop(0, CN, node, 0)
            pltpu.sync_copy(block_v, spmem.at[idx_v], add=True)
            return c2

        lax.fori_loop(0, NCH, chunk, 0)
        plsc.subcore_barrier()

        pltpu.sync_copy(
            spmem.at[pl.ds(rows0, NRT)],
            out_hbm.at[cid, pl.ds(rows0, NRT), pl.ds(p * FH, FH)])

        plsc.subcore_barrier()


def _t1_spmm(dst, vals, x):
    two = x.ndim == 3
    scratch = [
        pltpu.VMEM((CE,), jnp.int32),
        pltpu.VMEM((CE,), jnp.float32),
        pltpu.VMEM((CN, FH), jnp.float32),
    ]
    if two:
        scratch.append(pltpu.VMEM((CN, FH), jnp.float32))
    scratch += [
        pltpu.VMEM((CE, FH), jnp.float32),
        pltpu.VMEM((FH, FH), jnp.float32),
        pltpu.VMEM_SHARED((NPAD, FH), jnp.float32),
    ]
    k = pl.kernel(
        functools.partial(_t1_scatter_body, two),
        out_type=jax.ShapeDtypeStruct((2, NPAD, F), jnp.float32),
        mesh=_mesh(),
        compiler_params=pltpu.CompilerParams(needs_layout_passes=False),
        scratch_types=scratch,
    )
    return k(dst, vals, x)


def _tc1_body(x0, h1a, h1b, h11a, h11b, h2, h22, wt, w1, bt, bc, th_o, c1_o):
    a = [x0[...], h1a[...] + h1b[...], h11a[...] + h11b[...], h2[...],
         h22[...]]

    def mix(w_ref, nout):
        acc = jnp.zeros((a[0].shape[0], nout), jnp.float32)
        for m in range(5):
            acc = acc + jnp.dot(a[m], w_ref[m * LAT:(m + 1) * LAT, :],
                                preferred_element_type=jnp.float32)
        return acc

    th_o[...] = jax.nn.sigmoid(mix(wt, LAT) + bt[...])
    c1_o[...] = jnp.tanh(mix(w1, UNITS) + bc[...])


def _tc1(x0, h1a, h1b, h11a, h11b, h2, h22, wt, w1, bt, bc):
    blk16 = pl.BlockSpec((R, LAT), lambda i: (i, 0))
    wspec16 = pl.BlockSpec((5 * LAT, LAT), lambda i: (0, 0))
    wspec64 = pl.BlockSpec((5 * LAT, UNITS), lambda i: (0, 0))
    bspec16 = pl.BlockSpec((1, LAT), lambda i: (0, 0))
    bspec64 = pl.BlockSpec((1, UNITS), lambda i: (0, 0))
    return pl.pallas_call(
        _tc1_body,
        grid=(GRID,),
        in_specs=[blk16] * 7 + [wspec16, wspec64, bspec16, bspec64],
        out_specs=(blk16, pl.BlockSpec((R, UNITS), lambda i: (i, 0))),
        out_shape=(jax.ShapeDtypeStruct((M, LAT), jnp.float32),
                   jax.ShapeDtypeStruct((M, UNITS), jnp.float32)),
    )(x0, h1a, h1b, h11a, h11b, h2, h22, wt, w1, bt, bc)


def _tc2_body(c1, v0, v1, v2, v3, v4, p0_o, q1_o, q2_o, q3_o, q4_o):
    z = c1[...]
    outs = (p0_o, q1_o, q2_o, q3_o, q4_o)
    for o, v in zip(outs, (v0, v1, v2, v3, v4)):
        o[...] = jnp.dot(z, v[...], preferred_element_type=jnp.float32)


def _tc2(c1, vs):
    blk16 = pl.BlockSpec((R, LAT), lambda i: (i, 0))
    vspec = pl.BlockSpec((UNITS, LAT), lambda i: (0, 0))
    out16 = jax.ShapeDtypeStruct((M, LAT), jnp.float32)
    return pl.pallas_call(
        _tc2_body,
        grid=(GRID,),
        in_specs=[pl.BlockSpec((R, UNITS), lambda i: (i, 0))] + [vspec] * 5,
        out_specs=(blk16,) * 5,
        out_shape=(out16,) * 5,
    )(c1, *vs)


def _tc3_body(th, p0, r1a, r1b, r2a, r2b, r3, r4, bt, g_o):
    acc = (p0[...] + r1a[...] + r1b[...] + r2a[...] + r2b[...] + r3[...]
           + r4[...] + bt[...])
    g_o[...] = -th[...] * jnp.tanh(acc)


def _tc3(th, p0, r1a, r1b, r2a, r2b, r3, r4, bt):
    blk16 = pl.BlockSpec((R, LAT), lambda i: (i, 0))
    bspec16 = pl.BlockSpec((1, LAT), lambda i: (0, 0))
    return pl.pallas_call(
        _tc3_body,
        grid=(GRID,),
        in_specs=[blk16] * 8 + [bspec16],
        out_specs=blk16,
        out_shape=jax.ShapeDtypeStruct((M, LAT), jnp.float32),
    )(th, p0, r1a, r1b, r2a, r2b, r3, r4, bt)


def kernel(t_local, y, W_theta, W1, W2, b16, b64, sup1_idx, sup1_val,
           sup2_idx, sup2_val):
    x0 = y.reshape(B, N, LAT).transpose(1, 0, 2).reshape(N, F)
    x0 = jnp.pad(x0, ((0, NPAD - N), (0, 0)))
    pe = EPAD - N * DEG
    g_cols = jnp.pad(sup2_idx[1], (0, pe))
    g_vals = jnp.pad(sup2_val, (0, pe))
    s_dst = jnp.pad(sup1_idx[0], (0, pe))
    s_vals = jnp.pad(sup1_val, (0, pe))

    def fold(w, d):
        blocks = [w[m::5] for m in range(5)]
        return jnp.concatenate(
            [blocks[0] - blocks[2] - blocks[4], blocks[1], 2.0 * blocks[2],
             blocks[3], 2.0 * blocks[4]], axis=0)

    wt = fold(W_theta, LAT)
    w1 = fold(W1, LAT)
    v_blocks = [W2[m::5] for m in range(5)]
    vs = (v_blocks[0] - v_blocks[2] - v_blocks[4], v_blocks[1],
          2.0 * v_blocks[2], v_blocks[3], 2.0 * v_blocks[4])
    bt = b16.reshape(1, LAT)
    bc = b64.reshape(1, UNITS)

    h2 = _t2_spmm(g_cols, g_vals, x0)
    h22 = _t2_spmm(g_cols, g_vals, h2)
    h1 = _t1_spmm(s_dst, s_vals, x0)
    h11 = _t1_spmm(s_dst, s_vals, h1)
    h1a, h1b = h1[0], h1[1]
    h11a, h11b = h11[0], h11[1]

    r16 = lambda z: z.reshape(M, LAT)
    theta, c1 = _tc1(r16(x0), r16(h1a), r16(h1b), r16(h11a), r16(h11b),
                     r16(h2), r16(h22), wt, w1, bt, bc)

    p0, q1, q2, q3, q4 = _tc2(c1, vs)
    rf = lambda z: z.reshape(NPAD, F)

    r3 = _t2_spmm(g_cols, g_vals, rf(q3))
    s4 = _t2_spmm(g_cols, g_vals, rf(q4))
    r4 = _t2_spmm(g_cols, g_vals, s4)
    r1 = _t1_spmm(s_dst, s_vals, rf(q1))
    s2 = _t1_spmm(s_dst, s_vals, rf(q2))
    r2 = _t1_spmm(s_dst, s_vals, s2)

    g = _tc3(theta, p0, r16(r1[0]), r16(r1[1]), r16(r2[0]), r16(r2[1]),
             r16(r3), r16(r4), bt)
    return g.reshape(NPAD, B, LAT)[:N].transpose(1, 0, 2).reshape(B, N * LAT)


# speedup vs baseline: 3.4837x; 1.0423x over previous
import functools

import jax
import jax.numpy as jnp
from jax import lax
from jax.experimental import pallas as pl
from jax.experimental.pallas import tpu as pltpu
from jax.experimental.pallas import tpu_sc as plsc

N = 10000
DEG = 16
LAT = 16
UNITS = 64
B = 16
F = B * LAT
NC, NS, LN = 2, 16, 16
NT = NC * NS
NPAD = 10240
EPAD = NPAD * DEG
NPT = NPAD // NT
CN = 8
CE = CN * DEG
NCH = NPT // CN
FH = 32
NPASS = F // FH
CN2 = 4
CE2 = CN2 * DEG
NCH2 = NPT // CN2
NRT = NPAD // NS
M = NPAD * B
R = 2048
GRID = M // R


def _mesh():
    return plsc.VectorSubcoreMesh(
        core_axis_name="c", subcore_axis_name="s", num_cores=NC, num_subcores=NS
    )


def _t2_gather_body(cols_hbm, vals_hbm, x_hbm, out_hbm, colv, valv, rows0,
                    rows1, out_v, sem0, sem1):
    wid = lax.axis_index("s") * NC + lax.axis_index("c")
    nbase = wid * NPT
    ebase = nbase * DEG
    pltpu.sync_copy(cols_hbm.at[pl.ds(ebase, NPT * DEG)], colv)
    pltpu.sync_copy(vals_hbm.at[pl.ds(ebase, NPT * DEG)], valv)

    def issue(ch, rows, sem):
        return pltpu.async_copy(
            x_hbm.at[colv.at[pl.ds(ch * CE2, CE2)]], rows, sem)

    def compute(ch, rows):
        def node(n, c2):
            e = n * DEG
            eg = ch * CE2 + e
            accs = [jnp.zeros((LN,), jnp.float32) for _ in range(F // LN)]
            for j in range(DEG):
                vb = plsc.load_gather(
                    valv, [jnp.full((LN,), eg + j, jnp.int32)])
                for c in range(F // LN):
                    accs[c] = accs[c] + vb * rows[e + j, pl.ds(c * LN, LN)]
            for c in range(F // LN):
                out_v[n, pl.ds(c * LN, LN)] = accs[c]
            return c2

        lax.fori_loop(0, CN2, node, 0)
        pltpu.sync_copy(out_v, out_hbm.at[pl.ds(nbase + ch * CN2, CN2)])

    def pair(k, carry):
        ch0 = 2 * k
        d0 = issue(ch0, rows0, sem0)
        d1 = issue(ch0 + 1, rows1, sem1)
        d0.wait()
        compute(ch0, rows0)
        d1.wait()
        compute(ch0 + 1, rows1)
        return carry

    lax.fori_loop(0, NCH2 // 2, pair, 0)


def _t2_spmm(cols, vals, x):
    k = pl.kernel(
        _t2_gather_body,
        out_type=jax.ShapeDtypeStruct((NPAD, F), jnp.float32),
        mesh=_mesh(),
        compiler_params=pltpu.CompilerParams(needs_layout_passes=False),
        scratch_types=[
            pltpu.VMEM((NPT * DEG,), jnp.int32),
            pltpu.VMEM((NPT * DEG,), jnp.float32),
            pltpu.VMEM((CE2, F), jnp.float32),
            pltpu.VMEM((CE2, F), jnp.float32),
            pltpu.VMEM((CN2, F), jnp.float32),
            pltpu.SemaphoreType.DMA,
            pltpu.SemaphoreType.DMA,
        ],
    )
    return k(cols, vals, x)


FT1 = 128
NP1 = F // FT1


def _t1_scatter_body(two_parts, dst_hbm, vals_hbm, x_hbm, out_hbm, *rest):
    if two_parts:
        idx_v, vals_v, xrow_a, xrow_b, block_v, zbuf, spmem = rest
    else:
        idx_v, vals_v, xrow_a, block_v, zbuf, spmem = rest
        xrow_b = None
    cid = lax.axis_index("c")
    sid = lax.axis_index("s")
    wid = sid * NC + cid
    nbase = wid * NPT
    rows0 = sid * NRT

    def zrow(i, c2):
        for c in range(FT1 // LN):
            zbuf[i, pl.ds(c * LN, LN)] = jnp.zeros((LN,), jnp.float32)
        return c2

    lax.fori_loop(0, FT1, zrow, 0)

    for p in range(NP1):
        def zcp(kk, c2):
            pltpu.sync_copy(zbuf, spmem.at[pl.ds(rows0 + kk * FT1, FT1)])
            return c2

        lax.fori_loop(0, NRT // FT1, zcp, 0)
        plsc.subcore_barrier()

        def chunk(ch, c2):
            n0 = nbase + ch * CN
            e0 = n0 * DEG
            pltpu.sync_copy(dst_hbm.at[pl.ds(e0, CE)], idx_v)
            pltpu.sync_copy(vals_hbm.at[pl.ds(e0, CE)], vals_v)
            if two_parts:
                pltpu.sync_copy(
                    x_hbm.at[0, pl.ds(n0, CN), pl.ds(p * FT1, FT1)], xrow_a)
                pltpu.sync_copy(
                    x_hbm.at[1, pl.ds(n0, CN), pl.ds(p * FT1, FT1)], xrow_b)
            else:
                pltpu.sync_copy(
                    x_hbm.at[pl.ds(n0, CN), pl.ds(p * FT1, FT1)], xrow_a)

            def node(n, c3):
                e = n * DEG
                if two_parts:
                    rs = [xrow_a[n, pl.ds(c * LN, LN)]
                          + xrow_b[n, pl.ds(c * LN, LN)]
                          for c in range(FT1 // LN)]
                else:
                    rs = [xrow_a[n, pl.ds(c * LN, LN)]
                          for c in range(FT1 // LN)]
                for j in range(DEG):
                    vb = plsc.load_gather(
                        vals_v, [jnp.full((LN,), e + j, jnp.int32)])
                    for c in range(FT1 // LN):
                        block_v[e + j, pl.ds(c * LN, LN)] = vb * rs[c]
                return c3

            lax.fori_loop(0, CN, node, 0)
            pltpu.sync_copy(block_v, spmem.at[idx_v], add=True)
            return c2

        lax.fori_loop(0, NCH, chunk, 0)
        plsc.subcore_barrier()

        pltpu.sync_copy(
            spmem.at[pl.ds(rows0, NRT)],
            out_hbm.at[cid, pl.ds(rows0, NRT), pl.ds(p * FT1, FT1)])

        plsc.subcore_barrier()


def _t1_merge(raw):
    return raw


def _t1_spmm(dst, vals, x):
    two = x.ndim == 3
    scratch = [
        pltpu.VMEM((CE,), jnp.int32),
        pltpu.VMEM((CE,), jnp.float32),
        pltpu.VMEM((CN, FT1), jnp.float32),
    ]
    if two:
        scratch.append(pltpu.VMEM((CN, FT1), jnp.float32))
    scratch += [
        pltpu.VMEM((CE, FT1), jnp.float32),
        pltpu.VMEM((FT1, FT1), jnp.float32),
        pltpu.VMEM_SHARED((NPAD, FT1), jnp.float32),
    ]
    k = pl.kernel(
        functools.partial(_t1_scatter_body, two),
        out_type=jax.ShapeDtypeStruct((2, NPAD, F), jnp.float32),
        mesh=_mesh(),
        compiler_params=pltpu.CompilerParams(needs_layout_passes=False),
        scratch_types=scratch,
    )
    return k(dst, vals, x)


def _tc1_body(x0, h1a, h1b, h11a, h11b, h2, h22, wt, w1, bt, bc, th_o, c1_o):
    a = [x0[...], h1a[...] + h1b[...], h11a[...] + h11b[...], h2[...],
         h22[...]]

    def mix(w_ref, nout):
        acc = jnp.zeros((a[0].shape[0], nout), jnp.float32)
        for m in range(5):
            acc = acc + jnp.dot(a[m], w_ref[m * LAT:(m + 1) * LAT, :],
                                preferred_element_type=jnp.float32)
        return acc

    th_o[...] = jax.nn.sigmoid(mix(wt, LAT) + bt[...])
    c1_o[...] = jnp.tanh(mix(w1, UNITS) + bc[...])


def _tc1(x0, h1a, h1b, h11a, h11b, h2, h22, wt, w1, bt, bc):
    blk16 = pl.BlockSpec((R, LAT), lambda i: (i, 0))
    wspec16 = pl.BlockSpec((5 * LAT, LAT), lambda i: (0, 0))
    wspec64 = pl.BlockSpec((5 * LAT, UNITS), lambda i: (0, 0))
    bspec16 = pl.BlockSpec((1, LAT), lambda i: (0, 0))
    bspec64 = pl.BlockSpec((1, UNITS), lambda i: (0, 0))
    return pl.pallas_call(
        _tc1_body,
        grid=(GRID,),
        in_specs=[blk16] * 7 + [wspec16, wspec64, bspec16, bspec64],
        out_specs=(blk16, pl.BlockSpec((R, UNITS), lambda i: (i, 0))),
        out_shape=(jax.ShapeDtypeStruct((M, LAT), jnp.float32),
                   jax.ShapeDtypeStruct((M, UNITS), jnp.float32)),
    )(x0, h1a, h1b, h11a, h11b, h2, h22, wt, w1, bt, bc)


def _tc2_body(c1, v0, v1, v2, v3, v4, p0_o, q1_o, q2_o, q3_o, q4_o):
    z = c1[...]
    outs = (p0_o, q1_o, q2_o, q3_o, q4_o)
    for o, v in zip(outs, (v0, v1, v2, v3, v4)):
        o[...] = jnp.dot(z, v[...], preferred_element_type=jnp.float32)


def _tc2(c1, vs):
    blk16 = pl.BlockSpec((R, LAT), lambda i: (i, 0))
    vspec = pl.BlockSpec((UNITS, LAT), lambda i: (0, 0))
    out16 = jax.ShapeDtypeStruct((M, LAT), jnp.float32)
    return pl.pallas_call(
        _tc2_body,
        grid=(GRID,),
        in_specs=[pl.BlockSpec((R, UNITS), lambda i: (i, 0))] + [vspec] * 5,
        out_specs=(blk16,) * 5,
        out_shape=(out16,) * 5,
    )(c1, *vs)


def _tc3_body(th, p0, r1a, r1b, r2a, r2b, r3, r4, bt, g_o):
    acc = (p0[...] + r1a[...] + r1b[...] + r2a[...] + r2b[...] + r3[...]
           + r4[...] + bt[...])
    g_o[...] = -th[...] * jnp.tanh(acc)


def _tc3(th, p0, r1a, r1b, r2a, r2b, r3, r4, bt):
    blk16 = pl.BlockSpec((R, LAT), lambda i: (i, 0))
    bspec16 = pl.BlockSpec((1, LAT), lambda i: (0, 0))
    return pl.pallas_call(
        _tc3_body,
        grid=(GRID,),
        in_specs=[blk16] * 8 + [bspec16],
        out_specs=blk16,
        out_shape=jax.ShapeDtypeStruct((M, LAT), jnp.float32),
    )(th, p0, r1a, r1b, r2a, r2b, r3, r4, bt)


def kernel(t_local, y, W_theta, W1, W2, b16, b64, sup1_idx, sup1_val,
           sup2_idx, sup2_val):
    x0 = y.reshape(B, N, LAT).transpose(1, 0, 2).reshape(N, F)
    x0 = jnp.pad(x0, ((0, NPAD - N), (0, 0)))
    pe = EPAD - N * DEG
    g_cols = jnp.pad(sup2_idx[1], (0, pe))
    g_vals = jnp.pad(sup2_val, (0, pe))
    s_dst = jnp.pad(sup1_idx[0], (0, pe))
    s_vals = jnp.pad(sup1_val, (0, pe))

    def fold(w, d):
        blocks = [w[m::5] for m in range(5)]
        return jnp.concatenate(
            [blocks[0] - blocks[2] - blocks[4], blocks[1], 2.0 * blocks[2],
             blocks[3], 2.0 * blocks[4]], axis=0)

    wt = fold(W_theta, LAT)
    w1 = fold(W1, LAT)
    v_blocks = [W2[m::5] for m in range(5)]
    vs = (v_blocks[0] - v_blocks[2] - v_blocks[4], v_blocks[1],
          2.0 * v_blocks[2], v_blocks[3], 2.0 * v_blocks[4])
    bt = b16.reshape(1, LAT)
    bc = b64.reshape(1, UNITS)

    h2 = _t2_spmm(g_cols, g_vals, x0)
    h22 = _t2_spmm(g_cols, g_vals, h2)
    h1r = _t1_spmm(s_dst, s_vals, x0)
    h1 = _t1_merge(h1r)
    h11 = _t1_merge(_t1_spmm(s_dst, s_vals, h1r))
    h1a, h1b = h1[0], h1[1]
    h11a, h11b = h11[0], h11[1]

    r16 = lambda z: z.reshape(M, LAT)
    theta, c1 = _tc1(r16(x0), r16(h1a), r16(h1b), r16(h11a), r16(h11b),
                     r16(h2), r16(h22), wt, w1, bt, bc)

    p0, q1, q2, q3, q4 = _tc2(c1, vs)
    rf = lambda z: z.reshape(NPAD, F)

    r3 = _t2_spmm(g_cols, g_vals, rf(q3))
    s4 = _t2_spmm(g_cols, g_vals, rf(q4))
    r4 = _t2_spmm(g_cols, g_vals, s4)
    r1 = _t1_merge(_t1_spmm(s_dst, s_vals, rf(q1)))
    r2 = _t1_merge(_t1_spmm(s_dst, s_vals, _t1_spmm(s_dst, s_vals, rf(q2))))

    g = _tc3(theta, p0, r16(r1[0]), r16(r1[1]), r16(r2[0]), r16(r2[1]),
             r16(r3), r16(r4), bt)
    return g.reshape(NPAD, B, LAT)[:N].transpose(1, 0, 2).reshape(B, N * LAT)
